# Initial kernel scaffold; baseline (speedup 1.0000x reference)
import jax
import jax.numpy as jnp
from jax.experimental import pallas as pl


def kernel(cost_volume, peak_coords, mesh):
    raise NotImplementedError("write your pallas kernel here")


# speedup vs baseline: 28.8441x; 28.8441x over previous
import jax
import jax.numpy as jnp
from jax.experimental import pallas as pl

_H = 32
_W = 32
_HW = _H * _W
_M = 128


def _psrw_block_kernel(cv_ref, py_ref, px_ref, out_ref):
    cv = cv_ref[...]
    py = py_ref[...]
    px = px_ref[...]
    j = jax.lax.broadcasted_iota(jnp.int32, (_M, _HW), 1)
    y = j >> 5
    x = j & (_W - 1)
    dy = y - py
    dx = x - px
    d2 = dy * dy + dx * dx
    dist = jnp.sqrt(d2.astype(jnp.float32))

    near3 = (jnp.abs(dy) <= 1) & (jnp.abs(dx) <= 1)
    n3 = jnp.sum(near3.astype(jnp.float32), axis=1, keepdims=True)
    s_nm = jnp.sum(jnp.where(near3, 0.0, cv), axis=1, keepdims=True)
    cv_mean = s_nm / (float(_HW) - n3)

    mx = jnp.max(cv, axis=1, keepdims=True)

    compare = (cv <= cv_mean) & (d2 > 0)
    width = jnp.min(jnp.where(compare, dist, 100.0), axis=1, keepdims=True)
    radius = jnp.clip(width, 1.5, 4.5)

    disc = dist <= radius
    nsp = float(_HW) - jnp.sum(disc.astype(jnp.float32), axis=1, keepdims=True)
    s_side = jnp.sum(jnp.where(disc, 0.0, cv), axis=1, keepdims=True)
    mean_side = s_side / nsp
    resid = jnp.where(disc, 0.0, cv - mean_side)
    var_side = jnp.sum(resid * resid, axis=1, keepdims=True) / (nsp - 1.0)

    out_ref[...] = (mx - mean_side) / (var_side * width + 1e-16)


def _norm_kernel(p_ref, out_ref):
    p = p_ref[...]
    out_ref[...] = p / (jnp.mean(p, axis=1, keepdims=True) + 1e-8)


def kernel(cost_volume, peak_coords, mesh):
    B_, C_, H_, W_ = cost_volume.shape
    BC = B_ * C_
    cv = cost_volume.reshape(BC, H_ * W_)
    py = peak_coords[..., 0].reshape(BC, 1)
    px = peak_coords[..., 1].reshape(BC, 1)

    raw = pl.pallas_call(
        _psrw_block_kernel,
        grid=(BC // _M,),
        in_specs=[
            pl.BlockSpec((_M, H_ * W_), lambda i: (i, 0)),
            pl.BlockSpec((_M, 1), lambda i: (i, 0)),
            pl.BlockSpec((_M, 1), lambda i: (i, 0)),
        ],
        out_specs=pl.BlockSpec((_M, 1), lambda i: (i, 0)),
        out_shape=jax.ShapeDtypeStruct((BC, 1), jnp.float32),
    )(cv, py, px)

    psrw = raw.reshape(B_, C_)
    return pl.pallas_call(
        _norm_kernel,
        out_shape=jax.ShapeDtypeStruct((B_, C_), jnp.float32),
    )(psrw)
